# Initial kernel scaffold; baseline (speedup 1.0000x reference)
import jax
import jax.numpy as jnp
from jax.experimental import pallas as pl


def kernel(ids, embeddings, W, b):
    raise NotImplementedError("write your pallas kernel here")


# speedup vs baseline: 225.7671x; 225.7671x over previous
import functools

import jax
import jax.numpy as jnp
from jax import lax
from jax.experimental import pallas as pl
from jax.experimental.pallas import tpu as pltpu
from jax.experimental.pallas import tpu_sc as plsc

_L = 16
_NC = 2
_NS = 16


@functools.cache
def _build(B, S, V, D):
    NW = _NC * _NS
    rows_per_tile = B // NW
    NB = _L
    nblk = rows_per_tile // NB
    vrps = -(-V // _NS)
    vrps = -(-vrps // _L) * _L
    VP = _NS * vrps
    n_full = S // _L
    tail = S % _L
    assert B % NW == 0 and rows_per_tile % NB == 0
    assert D % _L == 0 and S >= _L and vrps <= V

    mesh = plsc.VectorSubcoreMesh(core_axis_name="c", subcore_axis_name="s")

    @functools.partial(
        pl.kernel,
        out_type=jax.ShapeDtypeStruct((B,), jnp.float32),
        mesh=mesh,
        compiler_params=pltpu.CompilerParams(needs_layout_passes=False),
        scratch_types=[
            pltpu.VMEM((NB, S), jnp.int32),
            pltpu.VMEM((NB, S), jnp.int32),
            pltpu.VMEM((VP,), jnp.float32),
            pltpu.VMEM((_L * _L,), jnp.float32),
            pltpu.VMEM((rows_per_tile,), jnp.float32),
            pltpu.VMEM((D,), jnp.float32),
            pltpu.VMEM((vrps, D), jnp.float32),
            pltpu.VMEM((vrps,), jnp.float32),
            pltpu.VMEM((_L,), jnp.float32),
            pltpu.VMEM_SHARED((VP,), jnp.float32),
            pltpu.SemaphoreType.DMA,
            pltpu.SemaphoreType.DMA,
        ],
    )
    def k(ids_hbm, emb_hbm, w_hbm, b_hbm, out_hbm,
          ids0, ids1, vtab, scr, outv, wv, ev, vst, bv, vsh, sem0, sem1):
        cid = lax.axis_index("c")
        sid = lax.axis_index("s")
        wid = sid * _NC + cid
        iota = lax.iota(jnp.int32, _L)
        tbase = iota * _L

        def transpose_sum():
            tot = plsc.load_gather(scr, [tbase])
            for t in range(1, _L):
                tot = tot + plsc.load_gather(scr, [tbase + t])
            return tot

        pltpu.sync_copy(w_hbm, wv)
        pltpu.sync_copy(b_hbm, bv)
        rem = V - (_NS - 1) * vrps
        eoff = sid * vrps

        @pl.when(sid < _NS - 1)
        def _():
            pltpu.sync_copy(emb_hbm.at[pl.ds(eoff, vrps)], ev)

        @pl.when(sid == _NS - 1)
        def _():
            pltpu.sync_copy(emb_hbm.at[pl.ds((_NS - 1) * vrps, rem)],
                            ev.at[pl.ds(0, rem)])

        for g in range(vrps // _L):
            for r in range(_L):
                p = ev[g * _L + r, pl.ds(0, _L)] * wv[pl.ds(0, _L)]
                for c in range(1, D // _L):
                    p = p + ev[g * _L + r, pl.ds(c * _L, _L)] * wv[pl.ds(c * _L, _L)]
                scr[pl.ds(r * _L, _L)] = p
            vst[pl.ds(g * _L, _L)] = transpose_sum()

        @pl.when(sid < _NS - 1)
        def _():
            pltpu.sync_copy(vst, vsh.at[pl.ds(eoff, vrps)])

        @pl.when(sid == _NS - 1)
        def _():
            pltpu.sync_copy(vst.at[pl.ds(0, rem)],
                            vsh.at[pl.ds((_NS - 1) * vrps, rem)])

        plsc.subcore_barrier()
        pltpu.sync_copy(vsh, vtab)

        base_row = wid * rows_per_tile
        zero = jnp.zeros((_L,), jnp.float32)
        tmask = iota >= (_L - tail)

        def blk_src(g):
            return ids_hbm.at[pl.ds(base_row + g * NB, NB)]

        pltpu.make_async_copy(blk_src(0), ids0, sem0).start()
        pltpu.make_async_copy(blk_src(1), ids1, sem1).start()

        def gather_row(buf, r, off):
            idv = buf[r, pl.ds(off, _L)]
            idv = jnp.minimum(jnp.maximum(idv, 0), V - 1)
            return plsc.load_gather(vtab, [idv])

        def body(i, carry):
            for slot, buf, sem in ((0, ids0, sem0), (1, ids1, sem1)):
                g = i * 2 + slot
                pltpu.make_async_copy(blk_src(g), buf, sem).wait()
                for r in range(NB):
                    acc = gather_row(buf, r, 0)
                    for c in range(1, n_full):
                        acc = acc + gather_row(buf, r, c * _L)
                    if tail:
                        gv = gather_row(buf, r, S - _L)
                        acc = acc + jnp.where(tmask, gv, zero)
                    scr[pl.ds(r * _L, _L)] = acc
                outv[pl.ds(g * NB, NB)] = transpose_sum() + bv[...]

                @pl.when(g + 2 < nblk)
                def _():
                    pltpu.make_async_copy(blk_src(g + 2), buf, sem).start()
            return carry

        lax.fori_loop(0, nblk // 2, body, 0)
        pltpu.sync_copy(outv, out_hbm.at[pl.ds(base_row, rows_per_tile)])

    return k


def kernel(ids, embeddings, W, b):
    B, S = ids.shape
    V, D = embeddings.shape
    out = _build(B, S, V, D)(
        ids,
        embeddings,
        W.reshape(D),
        jnp.broadcast_to(b.reshape(()), (_L,)),
    )
    return out.reshape(B, 1)


# speedup vs baseline: 226.6585x; 1.0039x over previous
import functools

import jax
import jax.numpy as jnp
from jax import lax
from jax.experimental import pallas as pl
from jax.experimental.pallas import tpu as pltpu
from jax.experimental.pallas import tpu_sc as plsc

_L = 16
_NC = 2
_NS = 16


@functools.cache
def _build(B, S, V, D):
    NW = _NC * _NS
    rows_per_tile = B // NW
    NB = _L
    nblk = rows_per_tile // NB
    vrps = -(-V // _NS)
    vrps = -(-vrps // _L) * _L
    VP = _NS * vrps
    n_full = S // _L
    tail = S % _L
    assert B % NW == 0 and rows_per_tile % NB == 0
    assert D % _L == 0 and S >= _L and vrps <= V

    mesh = plsc.VectorSubcoreMesh(core_axis_name="c", subcore_axis_name="s")

    @functools.partial(
        pl.kernel,
        out_type=jax.ShapeDtypeStruct((B,), jnp.float32),
        mesh=mesh,
        compiler_params=pltpu.CompilerParams(
            needs_layout_passes=False, use_tc_tiling_on_sc=True),
        scratch_types=[
            pltpu.VMEM((NB, S), jnp.int32),
            pltpu.VMEM((NB, S), jnp.int32),
            pltpu.VMEM((VP,), jnp.float32),
            pltpu.VMEM((_L * _L,), jnp.float32),
            pltpu.VMEM((rows_per_tile,), jnp.float32),
            pltpu.VMEM((D,), jnp.float32),
            pltpu.VMEM((vrps, D), jnp.float32),
            pltpu.VMEM((vrps,), jnp.float32),
            pltpu.VMEM((_L,), jnp.float32),
            pltpu.VMEM_SHARED((VP,), jnp.float32),
            pltpu.SemaphoreType.DMA,
            pltpu.SemaphoreType.DMA,
        ],
    )
    def k(ids_hbm, emb_hbm, w_hbm, b_hbm, out_hbm,
          ids0, ids1, vtab, scr, outv, wv, ev, vst, bv, vsh, sem0, sem1):
        cid = lax.axis_index("c")
        sid = lax.axis_index("s")
        wid = sid * _NC + cid
        iota = lax.iota(jnp.int32, _L)
        tbase = iota * _L

        def transpose_sum():
            tot = plsc.load_gather(scr, [tbase])
            for t in range(1, _L):
                tot = tot + plsc.load_gather(scr, [tbase + t])
            return tot

        pltpu.sync_copy(w_hbm, wv)
        pltpu.sync_copy(b_hbm, bv)
        rem = V - (_NS - 1) * vrps
        eoff = sid * vrps

        @pl.when(sid < _NS - 1)
        def _():
            pltpu.sync_copy(emb_hbm.at[pl.ds(eoff, vrps)], ev)

        @pl.when(sid == _NS - 1)
        def _():
            pltpu.sync_copy(emb_hbm.at[pl.ds((_NS - 1) * vrps, rem)],
                            ev.at[pl.ds(0, rem)])

        for g in range(vrps // _L):
            for r in range(_L):
                p = ev[g * _L + r, pl.ds(0, _L)] * wv[pl.ds(0, _L)]
                for c in range(1, D // _L):
                    p = p + ev[g * _L + r, pl.ds(c * _L, _L)] * wv[pl.ds(c * _L, _L)]
                scr[pl.ds(r * _L, _L)] = p
            vst[pl.ds(g * _L, _L)] = transpose_sum()

        @pl.when(sid < _NS - 1)
        def _():
            pltpu.sync_copy(vst, vsh.at[pl.ds(eoff, vrps)])

        @pl.when(sid == _NS - 1)
        def _():
            pltpu.sync_copy(vst.at[pl.ds(0, rem)],
                            vsh.at[pl.ds((_NS - 1) * vrps, rem)])

        plsc.subcore_barrier()
        pltpu.sync_copy(vsh, vtab)

        base_row = wid * rows_per_tile
        zero = jnp.zeros((_L,), jnp.float32)
        tmask = iota >= (_L - tail)

        def blk_src(g):
            return ids_hbm.at[pl.ds(base_row + g * NB, NB)]

        pltpu.make_async_copy(blk_src(0), ids0, sem0).start()
        pltpu.make_async_copy(blk_src(1), ids1, sem1).start()

        def gather_row(buf, r, off):
            idv = buf[r, pl.ds(off, _L)]
            idv = jnp.minimum(jnp.maximum(idv, 0), V - 1)
            return plsc.load_gather(vtab, [idv])

        def body(i, carry):
            for slot, buf, sem in ((0, ids0, sem0), (1, ids1, sem1)):
                g = i * 2 + slot
                pltpu.make_async_copy(blk_src(g), buf, sem).wait()
                for r in range(NB):
                    acc = gather_row(buf, r, 0)
                    for c in range(1, n_full):
                        acc = acc + gather_row(buf, r, c * _L)
                    if tail:
                        gv = gather_row(buf, r, S - _L)
                        acc = acc + jnp.where(tmask, gv, zero)
                    scr[pl.ds(r * _L, _L)] = acc
                outv[pl.ds(g * NB, NB)] = transpose_sum() + bv[...]

                @pl.when(g + 2 < nblk)
                def _():
                    pltpu.make_async_copy(blk_src(g + 2), buf, sem).start()
            return carry

        lax.fori_loop(0, nblk // 2, body, 0)
        pltpu.sync_copy(outv, out_hbm.at[pl.ds(base_row, rows_per_tile)])

    return k


def kernel(ids, embeddings, W, b):
    B, S = ids.shape
    V, D = embeddings.shape
    out = _build(B, S, V, D)(
        ids,
        embeddings,
        W.reshape(D),
        jnp.broadcast_to(b.reshape(()), (_L,)),
    )
    return out.reshape(B, 1)


# speedup vs baseline: 375.5332x; 1.6568x over previous
import functools

import jax
import jax.numpy as jnp
from jax import lax
from jax.experimental import pallas as pl
from jax.experimental.pallas import tpu as pltpu
from jax.experimental.pallas import tpu_sc as plsc

_L = 16
_NC = 2
_NS = 16


@functools.cache
def _build(B, S, V, D):
    NW = _NC * _NS
    rows_per_tile = B // NW
    CB = 8 * _L
    nblk = rows_per_tile // CB
    G = CB // _L
    vrps = -(-V // _NS)
    vrps = -(-vrps // _L) * _L
    VP = _NS * vrps
    assert B % (NW * CB) == 0 and D % _L == 0 and vrps <= V
    assert S % 2 == 0

    mesh = plsc.VectorSubcoreMesh(core_axis_name="c", subcore_axis_name="s")

    @functools.partial(
        pl.kernel,
        out_type=jax.ShapeDtypeStruct((B,), jnp.float32),
        mesh=mesh,
        compiler_params=pltpu.CompilerParams(
            needs_layout_passes=False, use_tc_tiling_on_sc=True),
        scratch_types=[
            pltpu.VMEM((S, CB), jnp.int32),
            pltpu.VMEM((S, CB), jnp.int32),
            pltpu.VMEM((VP,), jnp.float32),
            pltpu.VMEM((_L * _L,), jnp.float32),
            pltpu.VMEM((rows_per_tile,), jnp.float32),
            pltpu.VMEM((D,), jnp.float32),
            pltpu.VMEM((vrps, D), jnp.float32),
            pltpu.VMEM((vrps,), jnp.float32),
            pltpu.VMEM((_L,), jnp.float32),
            pltpu.VMEM_SHARED((VP,), jnp.float32),
            pltpu.SemaphoreType.DMA,
            pltpu.SemaphoreType.DMA,
        ],
    )
    def k(idst_hbm, emb_hbm, w_hbm, b_hbm, out_hbm,
          ids0, ids1, vtab, scr, outv, wv, ev, vst, bv, vsh, sem0, sem1):
        cid = lax.axis_index("c")
        sid = lax.axis_index("s")
        wid = sid * _NC + cid
        iota = lax.iota(jnp.int32, _L)
        tbase = iota * _L

        def transpose_sum():
            tot = plsc.load_gather(scr, [tbase])
            for t in range(1, _L):
                tot = tot + plsc.load_gather(scr, [tbase + t])
            return tot

        pltpu.sync_copy(w_hbm, wv)
        pltpu.sync_copy(b_hbm, bv)
        rem = V - (_NS - 1) * vrps
        eoff = sid * vrps

        @pl.when(sid < _NS - 1)
        def _():
            pltpu.sync_copy(emb_hbm.at[pl.ds(eoff, vrps)], ev)

        @pl.when(sid == _NS - 1)
        def _():
            pltpu.sync_copy(emb_hbm.at[pl.ds((_NS - 1) * vrps, rem)],
                            ev.at[pl.ds(0, rem)])

        for g in range(vrps // _L):
            for r in range(_L):
                p = ev[g * _L + r, pl.ds(0, _L)] * wv[pl.ds(0, _L)]
                for c in range(1, D // _L):
                    p = p + ev[g * _L + r, pl.ds(c * _L, _L)] * wv[pl.ds(c * _L, _L)]
                scr[pl.ds(r * _L, _L)] = p
            vst[pl.ds(g * _L, _L)] = transpose_sum()

        @pl.when(sid < _NS - 1)
        def _():
            pltpu.sync_copy(vst, vsh.at[pl.ds(eoff, vrps)])

        @pl.when(sid == _NS - 1)
        def _():
            pltpu.sync_copy(vst.at[pl.ds(0, rem)],
                            vsh.at[pl.ds((_NS - 1) * vrps, rem)])

        plsc.subcore_barrier()
        pltpu.sync_copy(vsh, vtab)

        base_row = wid * rows_per_tile
        zero = jnp.zeros((_L,), jnp.float32)

        def blk_src(g):
            return idst_hbm.at[pl.ds(0, S), pl.ds(base_row + g * CB, CB)]

        pltpu.make_async_copy(blk_src(0), ids0, sem0).start()
        pltpu.make_async_copy(blk_src(1), ids1, sem1).start()

        def gather16(buf, j, s):
            idv = buf[j, pl.ds(s * _L, _L)]
            idv = jnp.minimum(jnp.maximum(idv, 0), V - 1)
            return plsc.load_gather(vtab, [idv])

        for g in range(nblk):
            buf, sem = (ids0, sem0) if g % 2 == 0 else (ids1, sem1)
            pltpu.make_async_copy(blk_src(g), buf, sem).wait()

            def body(jh, accs, buf=buf):
                j = jh * 2
                accs = tuple(accs[s] + gather16(buf, j, s) for s in range(G))
                return tuple(accs[s] + gather16(buf, j + 1, s)
                             for s in range(G))

            accs = lax.fori_loop(0, S // 2, body, (zero,) * G, unroll=2)
            for s in range(G):
                outv[pl.ds(g * CB + s * _L, _L)] = accs[s] + bv[...]

            if g + 2 < nblk:
                pltpu.make_async_copy(blk_src(g + 2), buf, sem).start()

        pltpu.sync_copy(outv, out_hbm.at[pl.ds(base_row, rows_per_tile)])

    return k


def kernel(ids, embeddings, W, b):
    B, S = ids.shape
    V, D = embeddings.shape
    out = _build(B, S, V, D)(
        ids.T,
        embeddings,
        W.reshape(D),
        jnp.broadcast_to(b.reshape(()), (_L,)),
    )
    return out.reshape(B, 1)


# speedup vs baseline: 396.1820x; 1.0550x over previous
import functools

import jax
import jax.numpy as jnp
from jax import lax
from jax.experimental import pallas as pl
from jax.experimental.pallas import tpu as pltpu
from jax.experimental.pallas import tpu_sc as plsc

_L = 16
_NC = 2
_NS = 16


@functools.cache
def _build(B, S, V, D):
    NW = _NC * _NS
    rows_per_tile = B // NW
    CB = 8 * _L
    nblk = rows_per_tile // CB
    G = CB // _L
    vrps = -(-V // _NS)
    vrps = -(-vrps // _L) * _L
    VP = _NS * vrps
    assert B % (NW * CB) == 0 and D % _L == 0 and vrps <= V
    assert S % 2 == 0

    mesh = plsc.VectorSubcoreMesh(core_axis_name="c", subcore_axis_name="s")

    @functools.partial(
        pl.kernel,
        out_type=jax.ShapeDtypeStruct((B,), jnp.float32),
        mesh=mesh,
        compiler_params=pltpu.CompilerParams(
            needs_layout_passes=False, use_tc_tiling_on_sc=True),
        scratch_types=[
            pltpu.VMEM((S, CB), jnp.int32),
            pltpu.VMEM((S, CB), jnp.int32),
            pltpu.VMEM((VP,), jnp.float32),
            pltpu.VMEM((rows_per_tile,), jnp.float32),
            pltpu.VMEM((D,), jnp.float32),
            pltpu.VMEM((_L * _L,), jnp.float32),
            pltpu.VMEM((vrps, D), jnp.float32),
            pltpu.VMEM((vrps,), jnp.float32),
            pltpu.VMEM((_L,), jnp.float32),
            pltpu.VMEM_SHARED((VP,), jnp.float32),
            pltpu.SemaphoreType.DMA,
            pltpu.SemaphoreType.DMA,
        ],
    )
    def k(idst_hbm, emb_hbm, w_hbm, b_hbm, out_hbm,
          ids0, ids1, vtab, outv, wv, scr, ev, vst, bv, vsh, sem0, sem1):
        cid = lax.axis_index("c")
        sid = lax.axis_index("s")
        wid = sid * _NC + cid
        iota = lax.iota(jnp.int32, _L)
        tbase = iota * _L
        zero = jnp.zeros((_L,), jnp.float32)
        zero_i = jnp.zeros((_L,), jnp.int32)

        def transpose_sum():
            tot = plsc.load_gather(scr, [tbase])
            for t in range(1, _L):
                tot = tot + plsc.load_gather(scr, [tbase + t])
            return tot

        base_row = wid * rows_per_tile

        def blk_src(g):
            return idst_hbm.at[pl.ds(0, S), pl.ds(base_row + g * CB, CB)]

        pltpu.make_async_copy(blk_src(0), ids0, sem0).start()
        pltpu.make_async_copy(blk_src(1), ids1, sem1).start()

        pltpu.sync_copy(w_hbm, wv)
        pltpu.sync_copy(b_hbm, bv.at[pl.ds(0, 1)])
        bvec = plsc.load_gather(bv, [zero_i])
        rem = V - (_NS - 1) * vrps
        eoff = sid * vrps

        @pl.when(sid < _NS - 1)
        def _():
            pltpu.sync_copy(emb_hbm.at[pl.ds(eoff, vrps)], ev)

        @pl.when(sid == _NS - 1)
        def _():
            pltpu.sync_copy(emb_hbm.at[pl.ds((_NS - 1) * vrps, rem)],
                            ev.at[pl.ds(0, rem)])

        for g in range(vrps // _L):
            for r in range(_L):
                p = ev[g * _L + r, pl.ds(0, _L)] * wv[pl.ds(0, _L)]
                for c in range(1, D // _L):
                    p = p + ev[g * _L + r, pl.ds(c * _L, _L)] * wv[pl.ds(c * _L, _L)]
                scr[pl.ds(r * _L, _L)] = p
            vst[pl.ds(g * _L, _L)] = transpose_sum()

        @pl.when(sid < _NS - 1)
        def _():
            pltpu.sync_copy(vst, vsh.at[pl.ds(eoff, vrps)])

        @pl.when(sid == _NS - 1)
        def _():
            pltpu.sync_copy(vst.at[pl.ds(0, rem)],
                            vsh.at[pl.ds((_NS - 1) * vrps, rem)])

        plsc.subcore_barrier()
        pltpu.sync_copy(vsh, vtab)

        def gather16(buf, j, s):
            idv = buf[j, pl.ds(s * _L, _L)]
            idv = jnp.minimum(jnp.maximum(idv, 0), V - 1)
            return plsc.load_gather(vtab, [idv])

        for g in range(nblk):
            buf, sem = (ids0, sem0) if g % 2 == 0 else (ids1, sem1)
            pltpu.make_async_copy(blk_src(g), buf, sem).wait()

            def body(jh, accs, buf=buf):
                j = jh * 2
                accs = tuple(accs[s] + gather16(buf, j, s) for s in range(G))
                return tuple(accs[s] + gather16(buf, j + 1, s)
                             for s in range(G))

            accs = lax.fori_loop(0, S // 2, body, (zero,) * G, unroll=2)
            for s in range(G):
                outv[pl.ds(g * CB + s * _L, _L)] = accs[s] + bvec

            if g + 2 < nblk:
                pltpu.make_async_copy(blk_src(g + 2), buf, sem).start()

        pltpu.sync_copy(outv, out_hbm.at[pl.ds(base_row, rows_per_tile)])

    return k


def kernel(ids, embeddings, W, b):
    B, S = ids.shape
    V, D = embeddings.shape
    out = _build(B, S, V, D)(
        ids.T,
        embeddings,
        W.reshape(D),
        b.reshape(1),
    )
    return out.reshape(B, 1)
